# Initial kernel scaffold; baseline (speedup 1.0000x reference)
import jax
import jax.numpy as jnp
from jax.experimental import pallas as pl


def kernel(v, idx):
    raise NotImplementedError("write your pallas kernel here")


# speedup vs baseline: 1.8458x; 1.8458x over previous
import functools

import jax
import jax.numpy as jnp
from jax import lax
from jax.experimental import pallas as pl
from jax.experimental.pallas import tpu as pltpu
from jax.experimental.pallas import tpu_sc as plsc

_CHUNK = 1024


@functools.partial(jax.jit, static_argnames=("n_chunks", "chunk"))
def _gather_sc(v, idx_flat, n_chunks, chunk):
    n = idx_flat.shape[0]
    d = v.shape[1]
    info = plsc.get_sparse_core_info()
    nw = info.num_cores * info.num_subcores
    b_per_w = n // nw

    mesh = plsc.VectorSubcoreMesh(core_axis_name="c", subcore_axis_name="s")

    @functools.partial(
        pl.kernel,
        mesh=mesh,
        out_type=jax.ShapeDtypeStruct((n, d), jnp.float32),
        compiler_params=pltpu.CompilerParams(use_tc_tiling_on_sc=False),
        scratch_types=[
            pltpu.VMEM((chunk,), jnp.int32),
            pltpu.VMEM((chunk, d), jnp.float32),
            pltpu.SemaphoreType.DMA,
        ],
    )
    def k(table_hbm, idx_hbm, out_hbm, idx_v, rows_v, sem):
        wid = lax.axis_index("s") * info.num_cores + lax.axis_index("c")
        base = wid * b_per_w

        def body(g, carry):
            off = base + g * chunk
            pltpu.sync_copy(idx_hbm.at[pl.ds(off, chunk)], idx_v)
            pltpu.async_copy(table_hbm.at[idx_v], rows_v, sem).wait()
            pltpu.sync_copy(rows_v, out_hbm.at[pl.ds(off, chunk)])
            return carry

        lax.fori_loop(0, n_chunks, body, 0)

    return k(v, idx_flat)


def kernel(v, idx):
    b, h = idx.shape
    d = v.shape[1]
    n = b * h
    idx_flat = idx.reshape(n).astype(jnp.int32)
    n_chunks = n // (32 * _CHUNK)
    out = _gather_sc(v, idx_flat, n_chunks, _CHUNK)
    return out.reshape(b, h, d)


# speedup vs baseline: 1.8740x; 1.0153x over previous
import functools

import jax
import jax.numpy as jnp
from jax import lax
from jax.experimental import pallas as pl
from jax.experimental.pallas import tpu as pltpu
from jax.experimental.pallas import tpu_sc as plsc

_CHUNK = 640


@functools.partial(jax.jit, static_argnames=("n_chunks", "chunk"))
def _gather_sc(v, idx_flat, n_chunks, chunk):
    n = idx_flat.shape[0]
    d = v.shape[1]
    info = plsc.get_sparse_core_info()
    nw = info.num_cores * info.num_subcores
    b_per_w = n // nw

    mesh = plsc.VectorSubcoreMesh(core_axis_name="c", subcore_axis_name="s")

    @functools.partial(
        pl.kernel,
        mesh=mesh,
        out_type=jax.ShapeDtypeStruct((n, d), jnp.float32),
        compiler_params=pltpu.CompilerParams(use_tc_tiling_on_sc=False),
        scratch_types=[
            pltpu.VMEM((b_per_w,), jnp.int32),
            pltpu.VMEM((2, chunk, d), jnp.float32),
            pltpu.SemaphoreType.DMA,
            pltpu.SemaphoreType.DMA,
            pltpu.SemaphoreType.DMA,
            pltpu.SemaphoreType.DMA,
        ],
    )
    def k(table_hbm, idx_hbm, out_hbm, idx_all, rows_v, gsem0, gsem1,
          wsem0, wsem1):
        wid = lax.axis_index("s") * info.num_cores + lax.axis_index("c")
        base = wid * b_per_w
        pltpu.sync_copy(idx_hbm.at[pl.ds(base, b_per_w)], idx_all)

        gsem = (gsem0, gsem1)
        wsem = (wsem0, wsem1)

        def start_gather(g, slot):
            pltpu.make_async_copy(
                table_hbm.at[idx_all.at[pl.ds(g * chunk, chunk)]],
                rows_v.at[slot],
                gsem[slot],
            ).start()

        def wait_gather(slot):
            pltpu.make_async_copy(
                table_hbm.at[idx_all.at[pl.ds(0, chunk)]],
                rows_v.at[slot],
                gsem[slot],
            ).wait()

        def start_write(g, slot):
            pltpu.make_async_copy(
                rows_v.at[slot],
                out_hbm.at[pl.ds(base + g * chunk, chunk)],
                wsem[slot],
            ).start()

        def wait_write(slot):
            pltpu.make_async_copy(
                rows_v.at[slot],
                out_hbm.at[pl.ds(base, chunk)],
                wsem[slot],
            ).wait()

        start_gather(0, 0)

        start_gather(1, 1)
        wait_gather(0)
        start_write(0, 0)
        wait_write(0)
        start_gather(2, 0)
        wait_gather(1)
        start_write(1, 1)

        def body(i, carry):
            a = 2 * i
            wait_write(1)
            start_gather(a + 1, 1)
            wait_gather(0)
            start_write(a, 0)
            wait_write(0)
            start_gather(a + 2, 0)
            wait_gather(1)
            start_write(a + 1, 1)
            return carry

        lax.fori_loop(1, n_chunks // 2 - 1, body, 0)

        a = n_chunks - 2
        wait_write(1)
        start_gather(a + 1, 1)
        wait_gather(0)
        start_write(a, 0)
        wait_write(0)
        wait_gather(1)
        start_write(a + 1, 1)
        wait_write(1)

    return k(v, idx_flat)


def kernel(v, idx):
    b, h = idx.shape
    d = v.shape[1]
    n = b * h
    idx_flat = idx.reshape(n).astype(jnp.int32)
    n_chunks = n // (32 * _CHUNK)
    out = _gather_sc(v, idx_flat, n_chunks, _CHUNK)
    return out.reshape(b, h, d)
